# Initial kernel scaffold; baseline (speedup 1.0000x reference)
import jax
import jax.numpy as jnp
from jax.experimental import pallas as pl


def kernel(embedding_weights, sharded_sparse_features, sharded_offsets):
    raise NotImplementedError("write your pallas kernel here")


# speedup vs baseline: 14.1057x; 14.1057x over previous
import functools

import jax
import jax.numpy as jnp
import numpy as np
from jax import lax
from jax.experimental import pallas as pl
from jax.experimental.pallas import tpu as pltpu, tpu_sc as plsc


def _make_ebag(V, T, D, NB, L, NC, NS):
    NW = NC * NS
    BAGS_W = NB // NW
    G_BAGS = 104
    GROUPS = BAGS_W // G_BAGS
    CH = 80
    BAGS_CH = CH // L
    CHUNKS = (G_BAGS * L) // CH
    GE = G_BAGS * L

    mesh = plsc.VectorSubcoreMesh(core_axis_name="c", subcore_axis_name="s")

    @functools.partial(
        pl.kernel,
        out_type=jax.ShapeDtypeStruct((NB, D), jnp.float32),
        mesh=mesh,
        scratch_types=[
            pltpu.VMEM((GE,), jnp.int32),
            pltpu.VMEM((GE,), jnp.int32),
            pltpu.VMEM((CHUNKS, CH), jnp.int32),
            pltpu.VMEM((CH, D), jnp.float32),
            pltpu.VMEM((G_BAGS, D), jnp.float32),
            pltpu.SemaphoreType.DMA,
        ],
        compiler_params=pltpu.CompilerParams(use_tc_tiling_on_sc=False),
    )
    def ebag(table_hbm, idx_hbm, tbl_hbm, out_hbm,
             tbl_v, raw_v, flat_v, rows_v, out_v, sem):
        wid = lax.axis_index("s") * NC + lax.axis_index("c")
        w_elem = wid * (BAGS_W * L)
        w_bag = wid * BAGS_W

        pltpu.sync_copy(tbl_hbm, tbl_v)

        def group_body(g, carry):
            e_base = w_elem + g * GE
            pltpu.sync_copy(idx_hbm.at[pl.ds(e_base, GE)], raw_v)

            def idx_body(v, c2):
                raw = raw_v[pl.ds(v * 16, 16)]
                tbl = tbl_v[pl.ds(v * 16, 16)]
                flat_v[v // 5, pl.ds((v % 5) * 16, 16)] = raw * T + tbl
                return c2

            lax.fori_loop(0, GE // 16, idx_body, 0)

            def chunk_body(c, c2):
                pltpu.async_copy(table_hbm.at[flat_v.at[c]], rows_v, sem).wait()
                for k in range(BAGS_CH):
                    for h in range(D // 16):
                        acc = rows_v[k * L, pl.ds(h * 16, 16)]
                        for jj in range(1, L):
                            acc = acc + rows_v[k * L + jj, pl.ds(h * 16, 16)]
                        out_v[c * BAGS_CH + k, pl.ds(h * 16, 16)] = acc
                return c2

            lax.fori_loop(0, CHUNKS, chunk_body, 0)

            pltpu.sync_copy(out_v, out_hbm.at[pl.ds(w_bag + g * G_BAGS, G_BAGS)])
            return carry

        lax.fori_loop(0, GROUPS, group_body, 0)

    return ebag


def kernel(embedding_weights, sharded_sparse_features, sharded_offsets):
    V, T, D = embedding_weights.shape
    N = sharded_sparse_features.shape[0]
    NB = sharded_offsets.shape[0] - 1
    L = N // NB
    info = plsc.get_sparse_core_info()
    ebag = _make_ebag(V, T, D, NB, L, info.num_cores, info.num_subcores)
    table = embedding_weights.reshape(V * T, D)
    ge = 104 * L
    tbl_pat = jnp.asarray(
        np.tile(np.repeat(np.arange(T, dtype=np.int32), L), ge // (T * L)))
    out = ebag(table, sharded_sparse_features, tbl_pat)
    return out.reshape(NB // T, T, D)


# speedup vs baseline: 18.6010x; 1.3187x over previous
import functools

import jax
import jax.numpy as jnp
import numpy as np
from jax import lax
from jax.experimental import pallas as pl
from jax.experimental.pallas import tpu as pltpu, tpu_sc as plsc


def _make_ebag(V, T, D, NB, L, NC, NS):
    NW = NC * NS
    BAGS_W = NB // NW
    G_BAGS = 104
    GROUPS = BAGS_W // G_BAGS
    GE = G_BAGS * L
    CH = 80
    BAGS_CH = CH // L
    CHUNKS_W = BAGS_W * L // CH
    NBUF = 8
    OUT_CH = 2 * NBUF
    OUT_ROWS = NBUF * BAGS_CH
    OUTER = CHUNKS_W // OUT_CH

    mesh = plsc.VectorSubcoreMesh(core_axis_name="c", subcore_axis_name="s")

    @functools.partial(
        pl.kernel,
        out_type=jax.ShapeDtypeStruct((NB, D), jnp.float32),
        mesh=mesh,
        scratch_types=[
            pltpu.VMEM((GE,), jnp.int32),
            pltpu.VMEM((2, GE), jnp.int32),
            pltpu.VMEM((CHUNKS_W, CH), jnp.int32),
            pltpu.VMEM((NBUF, CH, D), jnp.float32),
            pltpu.VMEM((2, OUT_ROWS, D), jnp.float32),
            [pltpu.SemaphoreType.DMA] * NBUF,
            [pltpu.SemaphoreType.DMA] * 2,
            [pltpu.SemaphoreType.DMA] * 2,
        ],
        compiler_params=pltpu.CompilerParams(use_tc_tiling_on_sc=False),
    )
    def ebag(table_hbm, idx_hbm, tbl_hbm, out_hbm,
             tbl_v, raw_v, flat_v, rows_v, out_v, gsem, osem, rsem):
        wid = lax.axis_index("s") * NC + lax.axis_index("c")
        w_elem = wid * (BAGS_W * L)
        w_bag = wid * BAGS_W

        pltpu.sync_copy(tbl_hbm, tbl_v)

        def raw_copy(g, par):
            return pltpu.make_async_copy(
                idx_hbm.at[pl.ds(w_elem + g * GE, GE)], raw_v.at[par], rsem[par])

        raw_copy(0, 0).start()

        def group_body(gg, carry):
            for par in range(2):
                g = gg * 2 + par

                @pl.when(g + 1 < GROUPS)
                def _():
                    raw_copy(g + 1, 1 - par).start()

                raw_copy(g, par).wait()

                def idx_body(v, c2):
                    raw = raw_v[par, pl.ds(v * 16, 16)]
                    tbl = tbl_v[pl.ds(v * 16, 16)]
                    flat_v[g * (GE // CH) + v // 5, pl.ds((v % 5) * 16, 16)] = (
                        raw * T + tbl)
                    return c2

                lax.fori_loop(0, GE // 16, idx_body, 0, unroll=5)
            return carry

        lax.fori_loop(0, GROUPS // 2, group_body, 0)

        def gather(c, b):
            return pltpu.make_async_copy(
                table_hbm.at[flat_v.at[c]], rows_v.at[b], gsem[b])

        for b in range(NBUF):
            gather(b, b).start()

        def outer_body(c0, carry):
            cb = c0 * OUT_CH
            for half in range(2):
                @pl.when(c0 > 0)
                def _():
                    pltpu.make_async_copy(
                        out_v.at[half],
                        out_hbm.at[pl.ds(w_bag, OUT_ROWS)],
                        osem[half]).wait()

                for j8 in range(NBUF):
                    j = half * NBUF + j8
                    c = cb + j
                    b = j % NBUF
                    gather(c, b).wait()

                    def bag_body(k, c2):
                        base = k * L
                        for h in range(D // 16):
                            acc = rows_v[b, base, pl.ds(h * 16, 16)]
                            for jj in range(1, L):
                                acc = acc + rows_v[b, base + jj, pl.ds(h * 16, 16)]
                            out_v[half, j8 * BAGS_CH + k, pl.ds(h * 16, 16)] = acc
                        return c2

                    lax.fori_loop(0, BAGS_CH, bag_body, 0)

                    @pl.when(c + NBUF < CHUNKS_W)
                    def _():
                        gather(c + NBUF, b).start()

                pltpu.async_copy(
                    out_v.at[half],
                    out_hbm.at[pl.ds(w_bag + (cb + half * NBUF) * BAGS_CH,
                                     OUT_ROWS)],
                    osem[half])
            return carry

        lax.fori_loop(0, OUTER, outer_body, 0)

        for half in range(2):
            pltpu.make_async_copy(
                out_v.at[half], out_hbm.at[pl.ds(w_bag, OUT_ROWS)],
                osem[half]).wait()

    return ebag


def kernel(embedding_weights, sharded_sparse_features, sharded_offsets):
    V, T, D = embedding_weights.shape
    N = sharded_sparse_features.shape[0]
    NB = sharded_offsets.shape[0] - 1
    L = N // NB
    info = plsc.get_sparse_core_info()
    ebag = _make_ebag(V, T, D, NB, L, info.num_cores, info.num_subcores)
    table = embedding_weights.reshape(V * T, D)
    ge = 104 * L
    tbl_pat = jnp.asarray(
        np.tile(np.repeat(np.arange(T, dtype=np.int32), L), ge // (T * L)))
    out = ebag(table, sharded_sparse_features, tbl_pat)
    return out.reshape(NB // T, T, D)
